# baseline (speedup 1.0000x reference)
import jax
import jax.numpy as jnp
from jax.experimental import pallas as pl

_TOP_K = 2
_N_EXPERTS = 8
_BLK = 4096


def _gate_one(x, w, idx_ref, tw_ref):
    logits = jax.lax.dot_general(
        w, x, (((1,), (1,)), ((), ())), preferred_element_type=jnp.float32
    )

    iota = jax.lax.broadcasted_iota(jnp.int32, logits.shape, 0)
    l1 = jnp.max(logits, axis=0, keepdims=True)
    i1 = jnp.min(jnp.where(logits == l1, iota, _N_EXPERTS), axis=0, keepdims=True)
    masked = jnp.where(iota == i1, -jnp.inf, logits)
    l2 = jnp.max(masked, axis=0, keepdims=True)
    i2 = jnp.min(jnp.where(masked == l2, iota, _N_EXPERTS), axis=0, keepdims=True)

    unnorm = jnp.exp(logits - l1)
    denom = jnp.sum(unnorm, axis=0, keepdims=True)
    w1 = jnp.float32(1.0) / denom
    w2 = jnp.exp(l2 - l1) / denom

    idx_ref[...] = jnp.concatenate([i1, i2], axis=0).T
    tw_ref[...] = jnp.concatenate([w1, w2], axis=0).T


def _gate_body(x_ref, w_ref, idx_ref, tw_ref):
    _gate_one(x_ref[...], w_ref[...], idx_ref, tw_ref)


@jax.jit
def kernel(hidden_states, weight):
    bsz, seq_len, h = hidden_states.shape
    n = bsz * seq_len
    x = hidden_states.reshape(n, h)
    grid = (n // _BLK,)
    io_spec = pl.BlockSpec((_BLK, _TOP_K), lambda i: (i, 0))
    idx, tw = pl.pallas_call(
        _gate_body,
        grid=grid,
        in_specs=[
            pl.BlockSpec((_BLK, h), lambda i: (i, 0)),
            pl.BlockSpec((_N_EXPERTS, h), lambda i: (0, 0)),
        ],
        out_specs=[io_spec, io_spec],
        out_shape=[
            jax.ShapeDtypeStruct((n, _TOP_K), jnp.int32),
            jax.ShapeDtypeStruct((n, _TOP_K), jnp.float32),
        ],
    )(x, weight)
    return idx, tw


# speedup vs baseline: 1.8640x; 1.8640x over previous
import jax
import jax.numpy as jnp
from jax.experimental import pallas as pl

_TOP_K = 2
_N_EXPERTS = 8
_BLK = 4096


def _gate_one(x, w, idx_ref, tw_ref):
    logits = jax.lax.dot_general(
        w, x, (((1,), (1,)), ((), ())), preferred_element_type=jnp.float32
    )

    iota = jax.lax.broadcasted_iota(jnp.int32, logits.shape, 0)
    l1 = jnp.max(logits, axis=0, keepdims=True)
    i1 = jnp.min(jnp.where(logits == l1, iota, _N_EXPERTS), axis=0, keepdims=True)
    masked = jnp.where(iota == i1, -jnp.inf, logits)
    l2 = jnp.max(masked, axis=0, keepdims=True)
    i2 = jnp.min(jnp.where(masked == l2, iota, _N_EXPERTS), axis=0, keepdims=True)

    unnorm = jnp.exp(logits - l1)
    denom = jnp.sum(unnorm, axis=0, keepdims=True)
    w1 = jnp.float32(1.0) / denom
    w2 = jnp.exp(l2 - l1) / denom

    idx_ref[...] = jnp.concatenate([i1, i2], axis=0)
    tw_ref[...] = jnp.concatenate([w1, w2], axis=0)


def _gate_body(x_ref, w_ref, idx_ref, tw_ref):
    _gate_one(x_ref[...], w_ref[...], idx_ref, tw_ref)


@jax.jit
def kernel(hidden_states, weight):
    bsz, seq_len, h = hidden_states.shape
    n = bsz * seq_len
    x = hidden_states.reshape(n, h)
    grid = (n // _BLK,)
    io_spec = pl.BlockSpec((_TOP_K, _BLK), lambda i: (0, i))
    idx_t, tw_t = pl.pallas_call(
        _gate_body,
        grid=grid,
        in_specs=[
            pl.BlockSpec((_BLK, h), lambda i: (i, 0)),
            pl.BlockSpec((_N_EXPERTS, h), lambda i: (0, 0)),
        ],
        out_specs=[io_spec, io_spec],
        out_shape=[
            jax.ShapeDtypeStruct((_TOP_K, n), jnp.int32),
            jax.ShapeDtypeStruct((_TOP_K, n), jnp.float32),
        ],
    )(x, weight)
    return idx_t.T, tw_t.T
